# baseline (speedup 1.0000x reference)
import functools

import jax
import jax.numpy as jnp
from jax import lax
from jax.experimental import pallas as pl
from jax.experimental.pallas import tpu as pltpu
from jax.experimental.pallas import tpu_sc as plsc

N = 10000
E = 320000
H = 128
DE = 16
L = 3
NG = 128


_NC = 2
_NS = 16
_NW = _NC * _NS
_B = 128
_NBT = E // _B
_MAXB = (_NBT + _NW - 1) // _NW
_SPR = 624
_TAIL = N - _NS * _SPR

_C1 = -1.5957691216057308
_C2 = -0.07135481282006346


def _edge_body(h_hbm, ee_hbm, src_hbm, dst_hbm, em_hbm, out_hbm,
               src_v, dst_v, em_v, ee_v, hr_v, agg_sh, *sems):
    cid = lax.axis_index("c")
    sid = lax.axis_index("s")
    wid = cid * _NS + sid

    @plsc.parallel_loop(0, _SPR // 6)
    def _zero_row(r):
        for j in range(8):
            hr_v[r, pl.ds(16 * j, 16)] = jnp.zeros((16,), jnp.float32)

    r0 = sid * _SPR
    for k in range(6):
        pltpu.async_copy(hr_v.at[pl.ds(0, _SPR // 6)],
                         agg_sh.at[pl.ds(r0 + k * (_SPR // 6), _SPR // 6)],
                         sems[8])

    @pl.when(sid == _NS - 1)
    def _():
        pltpu.async_copy(hr_v.at[pl.ds(0, _TAIL)],
                         agg_sh.at[pl.ds(_NS * _SPR, _TAIL)], sems[8])
    for k in range(6):
        pltpu.make_async_copy(hr_v.at[pl.ds(0, _SPR // 6)],
                              agg_sh.at[pl.ds(r0, _SPR // 6)], sems[8]).wait()

    @pl.when(sid == _NS - 1)
    def _():
        pltpu.make_async_copy(hr_v.at[pl.ds(0, _TAIL)],
                              agg_sh.at[pl.ds(0, _TAIL)], sems[8]).wait()
    plsc.subcore_barrier()

    def _bi(k):
        return jnp.minimum(k * _NW + wid, _NBT - 1)

    def _issue_lin(k, b4):
        bi = _bi(k)
        pltpu.async_copy(src_hbm.at[pl.ds(bi, 1)],
                         src_v.at[pl.ds(b4, 1)], sems[b4])
        pltpu.async_copy(dst_hbm.at[pl.ds(bi, 1)],
                         dst_v.at[pl.ds(b4, 1)], sems[b4])

    def _wait_lin(b4):
        pltpu.make_async_copy(src_hbm.at[pl.ds(0, 1)],
                              src_v.at[pl.ds(b4, 1)], sems[b4]).wait()
        pltpu.make_async_copy(dst_hbm.at[pl.ds(0, 1)],
                              dst_v.at[pl.ds(b4, 1)], sems[b4]).wait()

    def _issue_eem(k, s2):
        bi = _bi(k)
        pltpu.async_copy(ee_hbm.at[pl.ds(bi * (_B // 2), _B // 2)],
                         ee_v.at[pl.ds(s2 * (_B // 2), _B // 2)], sems[4 + s2])
        pltpu.async_copy(em_hbm.at[pl.ds(bi, 1)],
                         em_v.at[pl.ds(s2, 1)], sems[4 + s2])

    def _wait_eem(s2):
        pltpu.make_async_copy(ee_hbm.at[pl.ds(0, _B // 2)],
                              ee_v.at[pl.ds(s2 * (_B // 2), _B // 2)],
                              sems[4 + s2]).wait()
        pltpu.make_async_copy(em_hbm.at[pl.ds(0, 1)],
                              em_v.at[pl.ds(s2, 1)], sems[4 + s2]).wait()

    def _issue_gather(s2, b4):
        pltpu.async_copy(h_hbm.at[src_v.at[b4]],
                         hr_v.at[pl.ds(s2 * _B, _B)], sems[6 + s2])

    def _wait_gather(s2):
        pltpu.make_async_copy(h_hbm.at[pl.ds(0, _B)],
                              hr_v.at[pl.ds(s2 * _B, _B)], sems[6 + s2]).wait()

    def _issue_scatter(s2, b4):
        pltpu.async_copy(hr_v.at[pl.ds(s2 * _B, _B)],
                         agg_sh.at[dst_v.at[b4]], sems[8 + s2], add=True)

    def _wait_scatter(s2):
        pltpu.make_async_copy(hr_v.at[pl.ds(s2 * _B, _B)],
                              agg_sh.at[pl.ds(0, _B)], sems[8 + s2]).wait()

    def _compute(k, s2):
        valid = (k * _NW + wid) < _NBT
        vmask = jnp.full((16,), jnp.where(valid, 1.0, 0.0), jnp.float32)

        @plsc.parallel_loop(0, _B, unroll=2)
        def _row(i):
            mv = em_v[s2, pl.ds((i // 16) * 16, 16)]
            lane = jnp.full((16,), 0, jnp.int32) + (i % 16)
            m = lax.gather(
                mv, lane[:, None],
                lax.GatherDimensionNumbers(
                    offset_dims=(), collapsed_slice_dims=(0,),
                    start_index_map=(0,)),
                (1,), mode=lax.GatherScatterMode.PROMISE_IN_BOUNDS) * vmask
            r = s2 * _B + i
            r2 = s2 * (_B // 2) + i // 2
            c0 = 64 * (i % 2)
            for q in range(4):
                ew = ee_v[r2, pl.ds(c0 + 16 * q, 16)]
                ea = lax.bitcast_convert_type(ew << 16, jnp.float32)
                eb = lax.bitcast_convert_type(
                    ew & jnp.int32(-65536), jnp.float32)
                for half, ex in ((0, ea), (1, eb)):
                    j8 = 2 * q + half
                    v = hr_v[r, pl.ds(16 * j8, 16)] + ex
                    t = v * (_C1 + _C2 * (v * v))
                    hr_v[r, pl.ds(16 * j8, 16)] = (v * m) / (1.0 + jnp.exp(t))

    NB = _MAXB

    _issue_lin(0, 0)
    _issue_lin(1, 1)
    _issue_eem(0, 0)
    _issue_eem(1, 1)
    _wait_lin(0)
    _issue_gather(0, 0)

    def _outer(k0, _):
        for b in range(4):
            k = k0 * 4 + b
            s2 = b & 1

            @pl.when(k < NB)
            def _():
                @pl.when(k + 1 < NB)
                def _():
                    _wait_lin((b + 1) % 4)

                    @pl.when(k >= 1)
                    def _():
                        _wait_scatter(1 - s2)
                    _issue_gather(1 - s2, (b + 1) % 4)

                @pl.when(k + 2 < NB)
                def _():
                    _issue_lin(k + 2, (b + 2) % 4)

                _wait_eem(s2)
                _wait_gather(s2)
                _compute(k, s2)
                _issue_scatter(s2, b)

                @pl.when(k + 2 < NB)
                def _():
                    _issue_eem(k + 2, s2)
        return 0
    lax.fori_loop(0, (NB + 3) // 4, _outer, 0)

    _wait_scatter((NB - 2) % 2)
    _wait_scatter((NB - 1) % 2)

    plsc.subcore_barrier()
    pltpu.sync_copy(agg_sh.at[pl.ds(r0, _SPR)],
                    out_hbm.at[cid, pl.ds(r0, _SPR)])

    @pl.when(sid == _NS - 1)
    def _():
        pltpu.sync_copy(agg_sh.at[pl.ds(_NS * _SPR, _TAIL)],
                        out_hbm.at[cid, pl.ds(_NS * _SPR, _TAIL)])


@functools.partial(jax.jit, static_argnames=())
def _edge_call(h, eeP, srcR, dstR, emR):
    mesh = plsc.VectorSubcoreMesh(core_axis_name="c", subcore_axis_name="s")
    f = pl.kernel(
        _edge_body,
        out_type=jax.ShapeDtypeStruct((_NC, N, H), jnp.float32),
        mesh=mesh,
        scratch_types=[
            pltpu.VMEM((4, 128), jnp.int32),
            pltpu.VMEM((4, 128), jnp.int32),
            pltpu.VMEM((2, 128), jnp.float32),
            pltpu.VMEM((_B, H), jnp.int32),
            pltpu.VMEM((2 * _B, H), jnp.float32),
            pltpu.VMEM_SHARED((N, H), jnp.float32),
        ] + [pltpu.SemaphoreType.DMA] * 10,
    )
    return f(h, eeP, srcR, dstR, emR)


def _gelu_tc(v):
    return 0.5 * v * (1.0 + lax.erf(v * 0.7071067811865476))


_BE = 6400


def _ee_body(ea_ref, wb_ref, bb_ref, o_ref):
    t = jnp.dot(ea_ref[...], wb_ref[...],
                preferred_element_type=jnp.float32) + bb_ref[...]
    a = jnp.concatenate([t[:, 0:16], t[:, 32:48], t[:, 64:80], t[:, 96:112]],
                        axis=1)
    b = jnp.concatenate([t[:, 16:32], t[:, 48:64], t[:, 80:96], t[:, 112:128]],
                        axis=1)
    a16 = lax.bitcast_convert_type(a.astype(jnp.bfloat16), jnp.uint16)
    b16 = lax.bitcast_convert_type(b.astype(jnp.bfloat16), jnp.uint16)
    o_ref[...] = ((b16.astype(jnp.uint32) << 16)
                  | a16.astype(jnp.uint32)).astype(jnp.int32)


def _ee_call(edge_attr, Wb, bb2):
    return pl.pallas_call(
        _ee_body,
        grid=(E // _BE,),
        in_specs=[
            pl.BlockSpec((_BE, DE), lambda i: (i, 0)),
            pl.BlockSpec((DE, H), lambda i: (0, 0)),
            pl.BlockSpec((1, H), lambda i: (0, 0)),
        ],
        out_specs=pl.BlockSpec((_BE, H // 2), lambda i: (i, 0)),
        out_shape=jax.ShapeDtypeStruct((E, H // 2), jnp.int32),
    )(edge_attr, Wb, bb2)


_BN = 2000


def _mlp_body(eps_ref, h_ref, a_ref, w1_ref, b1_ref, w2_ref, b2_ref,
              o_ref):
    t = eps_ref[0] * h_ref[...] + a_ref[0] + a_ref[1]
    u = _gelu_tc(jnp.dot(t, w1_ref[...], preferred_element_type=jnp.float32)
                 + b1_ref[...])
    v = jnp.dot(u, w2_ref[...], preferred_element_type=jnp.float32) + b2_ref[...]
    o_ref[...] = _gelu_tc(v)


def _mlp_call(epsp, h, parts, W1, b12, W2, b22):
    return pl.pallas_call(
        _mlp_body,
        grid=(N // _BN,),
        in_specs=[
            pl.BlockSpec(memory_space=pltpu.SMEM),
            pl.BlockSpec((_BN, H), lambda i: (i, 0)),
            pl.BlockSpec((_NC, _BN, H), lambda i: (0, i, 0)),
            pl.BlockSpec((H, H), lambda i: (0, 0)),
            pl.BlockSpec((1, H), lambda i: (0, 0)),
            pl.BlockSpec((H, H), lambda i: (0, 0)),
            pl.BlockSpec((1, H), lambda i: (0, 0)),
        ],
        out_specs=pl.BlockSpec((_BN, H), lambda i: (i, 0)),
        out_shape=jax.ShapeDtypeStruct((N, H), jnp.float32),
    )(epsp, h, parts, W1, b12, W2, b22)


def _pool_body(eps_ref, h_ref, a_ref, w1_ref, b1_ref, w2_ref, b2_ref,
               b_ref, m_ref, m1_ref, mb1_ref, m2_ref, mb2_ref,
               p_ref, num_sc, den_sc):
    i = pl.program_id(0)

    @pl.when(i == 0)
    def _():
        num_sc[...] = jnp.zeros_like(num_sc)
        den_sc[...] = jnp.zeros_like(den_sc)

    t0 = eps_ref[0] * h_ref[...] + a_ref[0] + a_ref[1]
    u0 = _gelu_tc(jnp.dot(t0, w1_ref[...], preferred_element_type=jnp.float32)
                  + b1_ref[...])
    hh = _gelu_tc(jnp.dot(u0, w2_ref[...], preferred_element_type=jnp.float32)
                  + b2_ref[...])
    t = jnp.dot(_gelu_tc(jnp.dot(hh, m1_ref[...],
                                 preferred_element_type=jnp.float32)
                         + mb1_ref[...]),
                m2_ref[...], preferred_element_type=jnp.float32) + mb2_ref[...]
    b = b_ref[0, 0, :]
    g = lax.broadcasted_iota(jnp.int32, (_BN, NG), 1)
    oh = (b[:, None] == g).astype(jnp.float32)
    mask = m_ref[0, 0, :][:, None]
    num_sc[...] += lax.dot_general(oh, t * mask, (((0,), (0,)), ((), ())))
    den_sc[...] += lax.dot_general(
        oh, jnp.broadcast_to(mask, (_BN, NG)), (((0,), (0,)), ((), ())))

    @pl.when(i == pl.num_programs(0) - 1)
    def _():
        p_ref[...] = num_sc[...] / (den_sc[...] + 1e-7)


def _pool_call(epsp, h, parts, W1, b12, W2, b22, batch3, mask3,
               M1, mb12, M2, mb22):
    return pl.pallas_call(
        _pool_body,
        grid=(N // _BN,),
        in_specs=[
            pl.BlockSpec(memory_space=pltpu.SMEM),
            pl.BlockSpec((_BN, H), lambda i: (i, 0)),
            pl.BlockSpec((_NC, _BN, H), lambda i: (0, i, 0)),
            pl.BlockSpec((H, H), lambda i: (0, 0)),
            pl.BlockSpec((1, H), lambda i: (0, 0)),
            pl.BlockSpec((H, H), lambda i: (0, 0)),
            pl.BlockSpec((1, H), lambda i: (0, 0)),
            pl.BlockSpec((1, 1, _BN), lambda i: (i, 0, 0)),
            pl.BlockSpec((1, 1, _BN), lambda i: (i, 0, 0)),
            pl.BlockSpec((H, H), lambda i: (0, 0)),
            pl.BlockSpec((1, H), lambda i: (0, 0)),
            pl.BlockSpec((H, H), lambda i: (0, 0)),
            pl.BlockSpec((1, H), lambda i: (0, 0)),
        ],
        out_specs=pl.BlockSpec((NG, H), lambda i: (0, 0)),
        out_shape=jax.ShapeDtypeStruct((NG, H), jnp.float32),
        scratch_shapes=[
            pltpu.VMEM((NG, H), jnp.float32),
            pltpu.VMEM((NG, NG), jnp.float32),
        ],
    )(epsp, h, parts, W1, b12, W2, b22, batch3, mask3, M1, mb12, M2, mb22)


def kernel(x, batch, edge_index, edge_attr, node_mask, edge_mask,
           conv_W1, conv_b1, conv_W2, conv_b2, eps, Wb, bb, M1, mb1, M2, mb2):
    srcR = edge_index[0].reshape(E // 128, 128)
    dstR = edge_index[1].reshape(E // 128, 128)
    emR = edge_mask.reshape(E // 128, 128)

    eeW = _ee_call(edge_attr, Wb, bb.reshape(1, H)).reshape(E // 2, H)

    h = x
    for i in range(L - 1):
        parts = _edge_call(h, eeW, srcR, dstR, emR)
        h = _mlp_call((1.0 + eps[i]).reshape(1), h, parts,
                      conv_W1[i], conv_b1[i].reshape(1, H),
                      conv_W2[i], conv_b2[i].reshape(1, H))

    parts = _edge_call(h, eeW, srcR, dstR, emR)
    return _pool_call((1.0 + eps[L - 1]).reshape(1), h, parts,
                      conv_W1[L - 1], conv_b1[L - 1].reshape(1, H),
                      conv_W2[L - 1], conv_b2[L - 1].reshape(1, H),
                      batch.reshape(N // _BN, 1, _BN),
                      node_mask.reshape(N // _BN, 1, _BN),
                      M1, mb1.reshape(1, H), M2, mb2.reshape(1, H))


# speedup vs baseline: 1.0312x; 1.0312x over previous
import functools

import jax
import jax.numpy as jnp
from jax import lax
from jax.experimental import pallas as pl
from jax.experimental.pallas import tpu as pltpu
from jax.experimental.pallas import tpu_sc as plsc

N = 10000
E = 320000
H = 128
DE = 16
L = 3
NG = 128


_NC = 2
_NS = 16
_NW = _NC * _NS
_B = 128
_NBT = E // _B
_MAXB = (_NBT + _NW - 1) // _NW
_SPR = 624
_TAIL = N - _NS * _SPR

_C1 = -1.5957691216057308
_C2 = -0.07135481282006346


def _edge_body(h_hbm, ee_hbm, src_hbm, dst_hbm, em_hbm, out_hbm,
               src_v, dst_v, em_v, ee_v, hr_v, agg_sh, *sems):
    cid = lax.axis_index("c")
    sid = lax.axis_index("s")
    wid = cid * _NS + sid

    @plsc.parallel_loop(0, _SPR // 6)
    def _zero_row(r):
        for j in range(8):
            hr_v[r, pl.ds(16 * j, 16)] = jnp.zeros((16,), jnp.float32)

    r0 = sid * _SPR
    for k in range(6):
        pltpu.async_copy(hr_v.at[pl.ds(0, _SPR // 6)],
                         agg_sh.at[pl.ds(r0 + k * (_SPR // 6), _SPR // 6)],
                         sems[8])

    @pl.when(sid == _NS - 1)
    def _():
        pltpu.async_copy(hr_v.at[pl.ds(0, _TAIL)],
                         agg_sh.at[pl.ds(_NS * _SPR, _TAIL)], sems[8])
    for k in range(6):
        pltpu.make_async_copy(hr_v.at[pl.ds(0, _SPR // 6)],
                              agg_sh.at[pl.ds(r0, _SPR // 6)], sems[8]).wait()

    @pl.when(sid == _NS - 1)
    def _():
        pltpu.make_async_copy(hr_v.at[pl.ds(0, _TAIL)],
                              agg_sh.at[pl.ds(0, _TAIL)], sems[8]).wait()
    plsc.subcore_barrier()

    def _bi(k):
        return jnp.minimum(k * _NW + wid, _NBT - 1)

    def _issue_lin(k, b4):
        bi = _bi(k)
        pltpu.async_copy(src_hbm.at[pl.ds(bi, 1)],
                         src_v.at[pl.ds(b4, 1)], sems[b4])
        pltpu.async_copy(dst_hbm.at[pl.ds(bi, 1)],
                         dst_v.at[pl.ds(b4, 1)], sems[b4])

    def _wait_lin(b4):
        pltpu.make_async_copy(src_hbm.at[pl.ds(0, 1)],
                              src_v.at[pl.ds(b4, 1)], sems[b4]).wait()
        pltpu.make_async_copy(dst_hbm.at[pl.ds(0, 1)],
                              dst_v.at[pl.ds(b4, 1)], sems[b4]).wait()

    def _issue_eem(k, s2):
        bi = _bi(k)
        pltpu.async_copy(ee_hbm.at[pl.ds(bi * (_B // 2), _B // 2)],
                         ee_v.at[pl.ds(s2 * (_B // 2), _B // 2)], sems[4 + s2])
        pltpu.async_copy(em_hbm.at[pl.ds(bi, 1)],
                         em_v.at[pl.ds(s2, 1)], sems[4 + s2])

    def _wait_eem(s2):
        pltpu.make_async_copy(ee_hbm.at[pl.ds(0, _B // 2)],
                              ee_v.at[pl.ds(s2 * (_B // 2), _B // 2)],
                              sems[4 + s2]).wait()
        pltpu.make_async_copy(em_hbm.at[pl.ds(0, 1)],
                              em_v.at[pl.ds(s2, 1)], sems[4 + s2]).wait()

    def _issue_gather(s2, b4):
        pltpu.async_copy(h_hbm.at[src_v.at[b4]],
                         hr_v.at[pl.ds(s2 * _B, _B)], sems[6 + s2])

    def _wait_gather(s2):
        pltpu.make_async_copy(h_hbm.at[pl.ds(0, _B)],
                              hr_v.at[pl.ds(s2 * _B, _B)], sems[6 + s2]).wait()

    def _issue_scatter(s2, b4):
        pltpu.async_copy(hr_v.at[pl.ds(s2 * _B, _B)],
                         agg_sh.at[dst_v.at[b4]], sems[8 + s2], add=True)

    def _wait_scatter(s2):
        pltpu.make_async_copy(hr_v.at[pl.ds(s2 * _B, _B)],
                              agg_sh.at[pl.ds(0, _B)], sems[8 + s2]).wait()

    def _compute(k, s2):
        valid = (k * _NW + wid) < _NBT
        vmask = jnp.full((16,), jnp.where(valid, 1.0, 0.0), jnp.float32)

        @plsc.parallel_loop(0, _B)
        def _row(i):
            mv = em_v[s2, pl.ds((i // 16) * 16, 16)]
            lane = jnp.full((16,), 0, jnp.int32) + (i % 16)
            m = lax.gather(
                mv, lane[:, None],
                lax.GatherDimensionNumbers(
                    offset_dims=(), collapsed_slice_dims=(0,),
                    start_index_map=(0,)),
                (1,), mode=lax.GatherScatterMode.PROMISE_IN_BOUNDS) * vmask
            r = s2 * _B + i
            r2 = s2 * (_B // 2) + i // 2
            c0 = 64 * (i % 2)
            for q in range(4):
                ew = ee_v[r2, pl.ds(c0 + 16 * q, 16)]
                ea = lax.bitcast_convert_type(ew << 16, jnp.float32)
                eb = lax.bitcast_convert_type(ew, jnp.float32)
                for half, ex in ((0, ea), (1, eb)):
                    j8 = 2 * q + half
                    v = hr_v[r, pl.ds(16 * j8, 16)] + ex
                    t = v * (_C1 + _C2 * (v * v))
                    hr_v[r, pl.ds(16 * j8, 16)] = (v * m) / (1.0 + jnp.exp(t))

    NB = _MAXB

    _issue_lin(0, 0)
    _issue_lin(1, 1)
    _issue_eem(0, 0)
    _issue_eem(1, 1)
    _wait_lin(0)
    _issue_gather(0, 0)

    def _outer(k0, _):
        for b in range(4):
            k = k0 * 4 + b
            s2 = b & 1

            @pl.when(k < NB)
            def _():
                @pl.when(k + 1 < NB)
                def _():
                    _wait_lin((b + 1) % 4)

                    @pl.when(k >= 1)
                    def _():
                        _wait_scatter(1 - s2)
                    _issue_gather(1 - s2, (b + 1) % 4)

                @pl.when(k + 2 < NB)
                def _():
                    _issue_lin(k + 2, (b + 2) % 4)

                _wait_eem(s2)
                _wait_gather(s2)
                _compute(k, s2)
                _issue_scatter(s2, b)

                @pl.when(k + 2 < NB)
                def _():
                    _issue_eem(k + 2, s2)
        return 0
    lax.fori_loop(0, (NB + 3) // 4, _outer, 0)

    _wait_scatter((NB - 2) % 2)
    _wait_scatter((NB - 1) % 2)

    plsc.subcore_barrier()
    pltpu.sync_copy(agg_sh.at[pl.ds(r0, _SPR)],
                    out_hbm.at[cid, pl.ds(r0, _SPR)])

    @pl.when(sid == _NS - 1)
    def _():
        pltpu.sync_copy(agg_sh.at[pl.ds(_NS * _SPR, _TAIL)],
                        out_hbm.at[cid, pl.ds(_NS * _SPR, _TAIL)])


@functools.partial(jax.jit, static_argnames=())
def _edge_call(h, eeP, srcR, dstR, emR):
    mesh = plsc.VectorSubcoreMesh(core_axis_name="c", subcore_axis_name="s")
    f = pl.kernel(
        _edge_body,
        out_type=jax.ShapeDtypeStruct((_NC, N, H), jnp.float32),
        mesh=mesh,
        scratch_types=[
            pltpu.VMEM((4, 128), jnp.int32),
            pltpu.VMEM((4, 128), jnp.int32),
            pltpu.VMEM((2, 128), jnp.float32),
            pltpu.VMEM((_B, H), jnp.int32),
            pltpu.VMEM((2 * _B, H), jnp.float32),
            pltpu.VMEM_SHARED((N, H), jnp.float32),
        ] + [pltpu.SemaphoreType.DMA] * 10,
    )
    return f(h, eeP, srcR, dstR, emR)


def _gelu_tc(v):
    return 0.5 * v * (1.0 + lax.erf(v * 0.7071067811865476))


_BE = 6400


def _ee_body(ea_ref, wb_ref, bb_ref, o_ref):
    t = jnp.dot(ea_ref[...], wb_ref[...],
                preferred_element_type=jnp.float32) + bb_ref[...]
    a = jnp.concatenate([t[:, 0:16], t[:, 32:48], t[:, 64:80], t[:, 96:112]],
                        axis=1)
    b = jnp.concatenate([t[:, 16:32], t[:, 48:64], t[:, 80:96], t[:, 112:128]],
                        axis=1)
    a16 = lax.bitcast_convert_type(a.astype(jnp.bfloat16), jnp.uint16)
    b16 = lax.bitcast_convert_type(b.astype(jnp.bfloat16), jnp.uint16)
    o_ref[...] = ((b16.astype(jnp.uint32) << 16)
                  | a16.astype(jnp.uint32)).astype(jnp.int32)


def _ee_call(edge_attr, Wb, bb2):
    return pl.pallas_call(
        _ee_body,
        grid=(E // _BE,),
        in_specs=[
            pl.BlockSpec((_BE, DE), lambda i: (i, 0)),
            pl.BlockSpec((DE, H), lambda i: (0, 0)),
            pl.BlockSpec((1, H), lambda i: (0, 0)),
        ],
        out_specs=pl.BlockSpec((_BE, H // 2), lambda i: (i, 0)),
        out_shape=jax.ShapeDtypeStruct((E, H // 2), jnp.int32),
    )(edge_attr, Wb, bb2)


_BN = 2000


def _mlp_body(eps_ref, h_ref, a_ref, w1_ref, b1_ref, w2_ref, b2_ref,
              o_ref):
    t = eps_ref[0] * h_ref[...] + a_ref[0] + a_ref[1]
    u = _gelu_tc(jnp.dot(t, w1_ref[...], preferred_element_type=jnp.float32)
                 + b1_ref[...])
    v = jnp.dot(u, w2_ref[...], preferred_element_type=jnp.float32) + b2_ref[...]
    o_ref[...] = _gelu_tc(v)


def _mlp_call(epsp, h, parts, W1, b12, W2, b22):
    return pl.pallas_call(
        _mlp_body,
        grid=(N // _BN,),
        in_specs=[
            pl.BlockSpec(memory_space=pltpu.SMEM),
            pl.BlockSpec((_BN, H), lambda i: (i, 0)),
            pl.BlockSpec((_NC, _BN, H), lambda i: (0, i, 0)),
            pl.BlockSpec((H, H), lambda i: (0, 0)),
            pl.BlockSpec((1, H), lambda i: (0, 0)),
            pl.BlockSpec((H, H), lambda i: (0, 0)),
            pl.BlockSpec((1, H), lambda i: (0, 0)),
        ],
        out_specs=pl.BlockSpec((_BN, H), lambda i: (i, 0)),
        out_shape=jax.ShapeDtypeStruct((N, H), jnp.float32),
    )(epsp, h, parts, W1, b12, W2, b22)


def _pool_body(eps_ref, h_ref, a_ref, w1_ref, b1_ref, w2_ref, b2_ref,
               b_ref, m_ref, m1_ref, mb1_ref, m2_ref, mb2_ref,
               p_ref, num_sc, den_sc):
    i = pl.program_id(0)

    @pl.when(i == 0)
    def _():
        num_sc[...] = jnp.zeros_like(num_sc)
        den_sc[...] = jnp.zeros_like(den_sc)

    t0 = eps_ref[0] * h_ref[...] + a_ref[0] + a_ref[1]
    u0 = _gelu_tc(jnp.dot(t0, w1_ref[...], preferred_element_type=jnp.float32)
                  + b1_ref[...])
    hh = _gelu_tc(jnp.dot(u0, w2_ref[...], preferred_element_type=jnp.float32)
                  + b2_ref[...])
    t = jnp.dot(_gelu_tc(jnp.dot(hh, m1_ref[...],
                                 preferred_element_type=jnp.float32)
                         + mb1_ref[...]),
                m2_ref[...], preferred_element_type=jnp.float32) + mb2_ref[...]
    b = b_ref[0, 0, :]
    g = lax.broadcasted_iota(jnp.int32, (_BN, NG), 1)
    oh = (b[:, None] == g).astype(jnp.float32)
    mask = m_ref[0, 0, :][:, None]
    num_sc[...] += lax.dot_general(oh, t * mask, (((0,), (0,)), ((), ())))
    den_sc[...] += lax.dot_general(
        oh, jnp.broadcast_to(mask, (_BN, NG)), (((0,), (0,)), ((), ())))

    @pl.when(i == pl.num_programs(0) - 1)
    def _():
        p_ref[...] = num_sc[...] / (den_sc[...] + 1e-7)


def _pool_call(epsp, h, parts, W1, b12, W2, b22, batch3, mask3,
               M1, mb12, M2, mb22):
    return pl.pallas_call(
        _pool_body,
        grid=(N // _BN,),
        in_specs=[
            pl.BlockSpec(memory_space=pltpu.SMEM),
            pl.BlockSpec((_BN, H), lambda i: (i, 0)),
            pl.BlockSpec((_NC, _BN, H), lambda i: (0, i, 0)),
            pl.BlockSpec((H, H), lambda i: (0, 0)),
            pl.BlockSpec((1, H), lambda i: (0, 0)),
            pl.BlockSpec((H, H), lambda i: (0, 0)),
            pl.BlockSpec((1, H), lambda i: (0, 0)),
            pl.BlockSpec((1, 1, _BN), lambda i: (i, 0, 0)),
            pl.BlockSpec((1, 1, _BN), lambda i: (i, 0, 0)),
            pl.BlockSpec((H, H), lambda i: (0, 0)),
            pl.BlockSpec((1, H), lambda i: (0, 0)),
            pl.BlockSpec((H, H), lambda i: (0, 0)),
            pl.BlockSpec((1, H), lambda i: (0, 0)),
        ],
        out_specs=pl.BlockSpec((NG, H), lambda i: (0, 0)),
        out_shape=jax.ShapeDtypeStruct((NG, H), jnp.float32),
        scratch_shapes=[
            pltpu.VMEM((NG, H), jnp.float32),
            pltpu.VMEM((NG, NG), jnp.float32),
        ],
    )(epsp, h, parts, W1, b12, W2, b22, batch3, mask3, M1, mb12, M2, mb22)


def kernel(x, batch, edge_index, edge_attr, node_mask, edge_mask,
           conv_W1, conv_b1, conv_W2, conv_b2, eps, Wb, bb, M1, mb1, M2, mb2):
    srcR = edge_index[0].reshape(E // 128, 128)
    dstR = edge_index[1].reshape(E // 128, 128)
    emR = edge_mask.reshape(E // 128, 128)

    eeW = _ee_call(edge_attr, Wb, bb.reshape(1, H)).reshape(E // 2, H)

    h = x
    for i in range(L - 1):
        parts = _edge_call(h, eeW, srcR, dstR, emR)
        h = _mlp_call((1.0 + eps[i]).reshape(1), h, parts,
                      conv_W1[i], conv_b1[i].reshape(1, H),
                      conv_W2[i], conv_b2[i].reshape(1, H))

    parts = _edge_call(h, eeW, srcR, dstR, emR)
    return _pool_call((1.0 + eps[L - 1]).reshape(1), h, parts,
                      conv_W1[L - 1], conv_b1[L - 1].reshape(1, H),
                      conv_W2[L - 1], conv_b2[L - 1].reshape(1, H),
                      batch.reshape(N // _BN, 1, _BN),
                      node_mask.reshape(N // _BN, 1, _BN),
                      M1, mb1.reshape(1, H), M2, mb2.reshape(1, H))


# speedup vs baseline: 1.1370x; 1.1026x over previous
import functools

import jax
import jax.numpy as jnp
from jax import lax
from jax.experimental import pallas as pl
from jax.experimental.pallas import tpu as pltpu
from jax.experimental.pallas import tpu_sc as plsc

N = 10000
E = 320000
H = 128
DE = 16
L = 3
NG = 128


_NC = 2
_NS = 16
_NW = _NC * _NS
_B = 128
_NBT = E // _B
_MAXB = (_NBT + _NW - 1) // _NW
_SPR = 624
_TAIL = N - _NS * _SPR

_C1 = -1.5957691216057308
_C2 = -0.07135481282006346


def _edge_body(h_hbm, ee_hbm, src_hbm, dst_hbm, em_hbm, out_hbm,
               src_v, dst_v, em_v, ee_v, hr_v, agg_sh, *sems):
    cid = lax.axis_index("c")
    sid = lax.axis_index("s")
    wid = cid * _NS + sid

    @plsc.parallel_loop(0, _SPR // 6)
    def _zero_row(r):
        for j in range(8):
            hr_v[r, pl.ds(16 * j, 16)] = jnp.zeros((16,), jnp.float32)

    r0 = sid * _SPR
    for k in range(6):
        pltpu.async_copy(hr_v.at[pl.ds(0, _SPR // 6)],
                         agg_sh.at[pl.ds(r0 + k * (_SPR // 6), _SPR // 6)],
                         sems[8])

    @pl.when(sid == _NS - 1)
    def _():
        pltpu.async_copy(hr_v.at[pl.ds(0, _TAIL)],
                         agg_sh.at[pl.ds(_NS * _SPR, _TAIL)], sems[8])
    for k in range(6):
        pltpu.make_async_copy(hr_v.at[pl.ds(0, _SPR // 6)],
                              agg_sh.at[pl.ds(r0, _SPR // 6)], sems[8]).wait()

    @pl.when(sid == _NS - 1)
    def _():
        pltpu.make_async_copy(hr_v.at[pl.ds(0, _TAIL)],
                              agg_sh.at[pl.ds(0, _TAIL)], sems[8]).wait()
    plsc.subcore_barrier()

    def _bi(k):
        return jnp.minimum(k * _NW + wid, _NBT - 1)

    def _issue_lin(k, b4):
        bi = _bi(k)
        pltpu.async_copy(src_hbm.at[pl.ds(bi, 1)],
                         src_v.at[pl.ds(b4, 1)], sems[b4])
        pltpu.async_copy(dst_hbm.at[pl.ds(bi, 1)],
                         dst_v.at[pl.ds(b4, 1)], sems[b4])

    def _wait_lin(b4):
        pltpu.make_async_copy(src_hbm.at[pl.ds(0, 1)],
                              src_v.at[pl.ds(b4, 1)], sems[b4]).wait()
        pltpu.make_async_copy(dst_hbm.at[pl.ds(0, 1)],
                              dst_v.at[pl.ds(b4, 1)], sems[b4]).wait()

    def _issue_eem(k, s2):
        bi = _bi(k)
        pltpu.async_copy(ee_hbm.at[pl.ds(bi * (_B // 2), _B // 2)],
                         ee_v.at[pl.ds(s2 * (_B // 2), _B // 2)], sems[4 + s2])
        pltpu.async_copy(em_hbm.at[pl.ds(bi, 1)],
                         em_v.at[pl.ds(s2, 1)], sems[4 + s2])

    def _wait_eem(s2):
        pltpu.make_async_copy(ee_hbm.at[pl.ds(0, _B // 2)],
                              ee_v.at[pl.ds(s2 * (_B // 2), _B // 2)],
                              sems[4 + s2]).wait()
        pltpu.make_async_copy(em_hbm.at[pl.ds(0, 1)],
                              em_v.at[pl.ds(s2, 1)], sems[4 + s2]).wait()

    def _issue_gather(s2, b4):
        pltpu.async_copy(h_hbm.at[src_v.at[b4]],
                         hr_v.at[pl.ds(s2 * _B, _B)], sems[6 + s2])

    def _wait_gather(s2):
        pltpu.make_async_copy(h_hbm.at[pl.ds(0, _B)],
                              hr_v.at[pl.ds(s2 * _B, _B)], sems[6 + s2]).wait()

    def _issue_scatter(s2, b4):
        pltpu.async_copy(hr_v.at[pl.ds(s2 * _B, _B)],
                         agg_sh.at[dst_v.at[b4]], sems[8 + s2], add=True)

    def _wait_scatter(s2):
        pltpu.make_async_copy(hr_v.at[pl.ds(s2 * _B, _B)],
                              agg_sh.at[pl.ds(0, _B)], sems[8 + s2]).wait()

    def _compute(k, s2):
        valid = (k * _NW + wid) < _NBT
        vmask = jnp.full((16,), jnp.where(valid, 1.0, 0.0), jnp.float32)

        @plsc.parallel_loop(0, _B)
        def _row(i):
            mv = em_v[s2, pl.ds((i // 16) * 16, 16)]
            lane = jnp.full((16,), 0, jnp.int32) + (i % 16)
            m = lax.gather(
                mv, lane[:, None],
                lax.GatherDimensionNumbers(
                    offset_dims=(), collapsed_slice_dims=(0,),
                    start_index_map=(0,)),
                (1,), mode=lax.GatherScatterMode.PROMISE_IN_BOUNDS) * vmask
            r = s2 * _B + i
            r2 = s2 * (_B // 2) + i // 2
            c0 = 64 * (i % 2)
            for q in range(4):
                ew = ee_v[r2, pl.ds(c0 + 16 * q, 16)]
                ea = lax.bitcast_convert_type(ew << 16, jnp.float32)
                eb = lax.bitcast_convert_type(ew, jnp.float32)
                for half, ex in ((0, ea), (1, eb)):
                    j8 = 2 * q + half
                    v = hr_v[r, pl.ds(16 * j8, 16)] + ex
                    t = v * (-1.702)
                    hr_v[r, pl.ds(16 * j8, 16)] = (v * m) / (1.0 + jnp.exp(t))

    NB = _MAXB

    _issue_lin(0, 0)
    _issue_lin(1, 1)
    _issue_eem(0, 0)
    _issue_eem(1, 1)
    _wait_lin(0)
    _issue_gather(0, 0)

    def _outer(k0, _):
        for b in range(4):
            k = k0 * 4 + b
            s2 = b & 1

            @pl.when(k < NB)
            def _():
                @pl.when(k + 1 < NB)
                def _():
                    _wait_lin((b + 1) % 4)

                    @pl.when(k >= 1)
                    def _():
                        _wait_scatter(1 - s2)
                    _issue_gather(1 - s2, (b + 1) % 4)

                @pl.when(k + 2 < NB)
                def _():
                    _issue_lin(k + 2, (b + 2) % 4)

                _wait_eem(s2)
                _wait_gather(s2)
                _compute(k, s2)
                _issue_scatter(s2, b)

                @pl.when(k + 2 < NB)
                def _():
                    _issue_eem(k + 2, s2)
        return 0
    lax.fori_loop(0, (NB + 3) // 4, _outer, 0)

    _wait_scatter((NB - 2) % 2)
    _wait_scatter((NB - 1) % 2)

    plsc.subcore_barrier()
    pltpu.sync_copy(agg_sh.at[pl.ds(r0, _SPR)],
                    out_hbm.at[cid, pl.ds(r0, _SPR)])

    @pl.when(sid == _NS - 1)
    def _():
        pltpu.sync_copy(agg_sh.at[pl.ds(_NS * _SPR, _TAIL)],
                        out_hbm.at[cid, pl.ds(_NS * _SPR, _TAIL)])


@functools.partial(jax.jit, static_argnames=())
def _edge_call(h, eeP, srcR, dstR, emR):
    mesh = plsc.VectorSubcoreMesh(core_axis_name="c", subcore_axis_name="s")
    f = pl.kernel(
        _edge_body,
        out_type=jax.ShapeDtypeStruct((_NC, N, H), jnp.float32),
        mesh=mesh,
        scratch_types=[
            pltpu.VMEM((4, 128), jnp.int32),
            pltpu.VMEM((4, 128), jnp.int32),
            pltpu.VMEM((2, 128), jnp.float32),
            pltpu.VMEM((_B, H), jnp.int32),
            pltpu.VMEM((2 * _B, H), jnp.float32),
            pltpu.VMEM_SHARED((N, H), jnp.float32),
        ] + [pltpu.SemaphoreType.DMA] * 10,
    )
    return f(h, eeP, srcR, dstR, emR)


def _gelu_tc(v):
    return 0.5 * v * (1.0 + lax.erf(v * 0.7071067811865476))


_BE = 6400


def _ee_body(ea_ref, wb_ref, bb_ref, o_ref):
    t = jnp.dot(ea_ref[...], wb_ref[...],
                preferred_element_type=jnp.float32) + bb_ref[...]
    a = jnp.concatenate([t[:, 0:16], t[:, 32:48], t[:, 64:80], t[:, 96:112]],
                        axis=1)
    b = jnp.concatenate([t[:, 16:32], t[:, 48:64], t[:, 80:96], t[:, 112:128]],
                        axis=1)
    a16 = lax.bitcast_convert_type(a.astype(jnp.bfloat16), jnp.uint16)
    b16 = lax.bitcast_convert_type(b.astype(jnp.bfloat16), jnp.uint16)
    o_ref[...] = ((b16.astype(jnp.uint32) << 16)
                  | a16.astype(jnp.uint32)).astype(jnp.int32)


def _ee_call(edge_attr, Wb, bb2):
    return pl.pallas_call(
        _ee_body,
        grid=(E // _BE,),
        in_specs=[
            pl.BlockSpec((_BE, DE), lambda i: (i, 0)),
            pl.BlockSpec((DE, H), lambda i: (0, 0)),
            pl.BlockSpec((1, H), lambda i: (0, 0)),
        ],
        out_specs=pl.BlockSpec((_BE, H // 2), lambda i: (i, 0)),
        out_shape=jax.ShapeDtypeStruct((E, H // 2), jnp.int32),
    )(edge_attr, Wb, bb2)


_BN = 2000


def _mlp_body(eps_ref, h_ref, a_ref, w1_ref, b1_ref, w2_ref, b2_ref,
              o_ref):
    t = eps_ref[0] * h_ref[...] + a_ref[0] + a_ref[1]
    u = _gelu_tc(jnp.dot(t, w1_ref[...], preferred_element_type=jnp.float32)
                 + b1_ref[...])
    v = jnp.dot(u, w2_ref[...], preferred_element_type=jnp.float32) + b2_ref[...]
    o_ref[...] = _gelu_tc(v)


def _mlp_call(epsp, h, parts, W1, b12, W2, b22):
    return pl.pallas_call(
        _mlp_body,
        grid=(N // _BN,),
        in_specs=[
            pl.BlockSpec(memory_space=pltpu.SMEM),
            pl.BlockSpec((_BN, H), lambda i: (i, 0)),
            pl.BlockSpec((_NC, _BN, H), lambda i: (0, i, 0)),
            pl.BlockSpec((H, H), lambda i: (0, 0)),
            pl.BlockSpec((1, H), lambda i: (0, 0)),
            pl.BlockSpec((H, H), lambda i: (0, 0)),
            pl.BlockSpec((1, H), lambda i: (0, 0)),
        ],
        out_specs=pl.BlockSpec((_BN, H), lambda i: (i, 0)),
        out_shape=jax.ShapeDtypeStruct((N, H), jnp.float32),
    )(epsp, h, parts, W1, b12, W2, b22)


def _pool_body(eps_ref, h_ref, a_ref, w1_ref, b1_ref, w2_ref, b2_ref,
               b_ref, m_ref, m1_ref, mb1_ref, m2_ref, mb2_ref,
               p_ref, num_sc, den_sc):
    i = pl.program_id(0)

    @pl.when(i == 0)
    def _():
        num_sc[...] = jnp.zeros_like(num_sc)
        den_sc[...] = jnp.zeros_like(den_sc)

    t0 = eps_ref[0] * h_ref[...] + a_ref[0] + a_ref[1]
    u0 = _gelu_tc(jnp.dot(t0, w1_ref[...], preferred_element_type=jnp.float32)
                  + b1_ref[...])
    hh = _gelu_tc(jnp.dot(u0, w2_ref[...], preferred_element_type=jnp.float32)
                  + b2_ref[...])
    t = jnp.dot(_gelu_tc(jnp.dot(hh, m1_ref[...],
                                 preferred_element_type=jnp.float32)
                         + mb1_ref[...]),
                m2_ref[...], preferred_element_type=jnp.float32) + mb2_ref[...]
    b = b_ref[0, 0, :]
    g = lax.broadcasted_iota(jnp.int32, (_BN, NG), 1)
    oh = (b[:, None] == g).astype(jnp.float32)
    mask = m_ref[0, 0, :][:, None]
    num_sc[...] += lax.dot_general(oh, t * mask, (((0,), (0,)), ((), ())))
    den_sc[...] += lax.dot_general(
        oh, jnp.broadcast_to(mask, (_BN, NG)), (((0,), (0,)), ((), ())))

    @pl.when(i == pl.num_programs(0) - 1)
    def _():
        p_ref[...] = num_sc[...] / (den_sc[...] + 1e-7)


def _pool_call(epsp, h, parts, W1, b12, W2, b22, batch3, mask3,
               M1, mb12, M2, mb22):
    return pl.pallas_call(
        _pool_body,
        grid=(N // _BN,),
        in_specs=[
            pl.BlockSpec(memory_space=pltpu.SMEM),
            pl.BlockSpec((_BN, H), lambda i: (i, 0)),
            pl.BlockSpec((_NC, _BN, H), lambda i: (0, i, 0)),
            pl.BlockSpec((H, H), lambda i: (0, 0)),
            pl.BlockSpec((1, H), lambda i: (0, 0)),
            pl.BlockSpec((H, H), lambda i: (0, 0)),
            pl.BlockSpec((1, H), lambda i: (0, 0)),
            pl.BlockSpec((1, 1, _BN), lambda i: (i, 0, 0)),
            pl.BlockSpec((1, 1, _BN), lambda i: (i, 0, 0)),
            pl.BlockSpec((H, H), lambda i: (0, 0)),
            pl.BlockSpec((1, H), lambda i: (0, 0)),
            pl.BlockSpec((H, H), lambda i: (0, 0)),
            pl.BlockSpec((1, H), lambda i: (0, 0)),
        ],
        out_specs=pl.BlockSpec((NG, H), lambda i: (0, 0)),
        out_shape=jax.ShapeDtypeStruct((NG, H), jnp.float32),
        scratch_shapes=[
            pltpu.VMEM((NG, H), jnp.float32),
            pltpu.VMEM((NG, NG), jnp.float32),
        ],
    )(epsp, h, parts, W1, b12, W2, b22, batch3, mask3, M1, mb12, M2, mb22)


def kernel(x, batch, edge_index, edge_attr, node_mask, edge_mask,
           conv_W1, conv_b1, conv_W2, conv_b2, eps, Wb, bb, M1, mb1, M2, mb2):
    srcR = edge_index[0].reshape(E // 128, 128)
    dstR = edge_index[1].reshape(E // 128, 128)
    emR = edge_mask.reshape(E // 128, 128)

    eeW = _ee_call(edge_attr, Wb, bb.reshape(1, H)).reshape(E // 2, H)

    h = x
    for i in range(L - 1):
        parts = _edge_call(h, eeW, srcR, dstR, emR)
        h = _mlp_call((1.0 + eps[i]).reshape(1), h, parts,
                      conv_W1[i], conv_b1[i].reshape(1, H),
                      conv_W2[i], conv_b2[i].reshape(1, H))

    parts = _edge_call(h, eeW, srcR, dstR, emR)
    return _pool_call((1.0 + eps[L - 1]).reshape(1), h, parts,
                      conv_W1[L - 1], conv_b1[L - 1].reshape(1, H),
                      conv_W2[L - 1], conv_b2[L - 1].reshape(1, H),
                      batch.reshape(N // _BN, 1, _BN),
                      node_mask.reshape(N // _BN, 1, _BN),
                      M1, mb1.reshape(1, H), M2, mb2.reshape(1, H))


# speedup vs baseline: 1.1400x; 1.0026x over previous
import functools

import jax
import jax.numpy as jnp
from jax import lax
from jax.experimental import pallas as pl
from jax.experimental.pallas import tpu as pltpu
from jax.experimental.pallas import tpu_sc as plsc

N = 10000
E = 320000
H = 128
DE = 16
L = 3
NG = 128


_NC = 2
_NS = 16
_NW = _NC * _NS
_B = 128
_NBT = E // _B
_MAXB = (_NBT + _NW - 1) // _NW
_SPR = 624
_TAIL = N - _NS * _SPR

_C1 = -1.5957691216057308
_C2 = -0.07135481282006346


def _edge_body(h_hbm, ee_hbm, src_hbm, dst_hbm, em_hbm, out_hbm,
               src_v, dst_v, em_v, ee_v, hr_v, agg_sh, *sems):
    cid = lax.axis_index("c")
    sid = lax.axis_index("s")
    wid = cid * _NS + sid

    @plsc.parallel_loop(0, _SPR // 6)
    def _zero_row(r):
        for j in range(8):
            hr_v[r, pl.ds(16 * j, 16)] = jnp.zeros((16,), jnp.float32)

    r0 = sid * _SPR
    for k in range(6):
        pltpu.async_copy(hr_v.at[pl.ds(0, _SPR // 6)],
                         agg_sh.at[pl.ds(r0 + k * (_SPR // 6), _SPR // 6)],
                         sems[8])

    @pl.when(sid == _NS - 1)
    def _():
        pltpu.async_copy(hr_v.at[pl.ds(0, _TAIL)],
                         agg_sh.at[pl.ds(_NS * _SPR, _TAIL)], sems[8])
    for k in range(6):
        pltpu.make_async_copy(hr_v.at[pl.ds(0, _SPR // 6)],
                              agg_sh.at[pl.ds(r0, _SPR // 6)], sems[8]).wait()

    @pl.when(sid == _NS - 1)
    def _():
        pltpu.make_async_copy(hr_v.at[pl.ds(0, _TAIL)],
                              agg_sh.at[pl.ds(0, _TAIL)], sems[8]).wait()
    plsc.subcore_barrier()

    def _bi(k):
        return jnp.minimum(k * _NW + wid, _NBT - 1)

    def _issue_lin(k, b4):
        bi = _bi(k)
        pltpu.async_copy(src_hbm.at[pl.ds(bi, 1)],
                         src_v.at[pl.ds(b4, 1)], sems[b4])
        pltpu.async_copy(dst_hbm.at[pl.ds(bi, 1)],
                         dst_v.at[pl.ds(b4, 1)], sems[b4])

    def _wait_lin(b4):
        pltpu.make_async_copy(src_hbm.at[pl.ds(0, 1)],
                              src_v.at[pl.ds(b4, 1)], sems[b4]).wait()
        pltpu.make_async_copy(dst_hbm.at[pl.ds(0, 1)],
                              dst_v.at[pl.ds(b4, 1)], sems[b4]).wait()

    def _issue_eem(k, s2):
        bi = _bi(k)
        pltpu.async_copy(ee_hbm.at[pl.ds(bi * (_B // 2), _B // 2)],
                         ee_v.at[pl.ds(s2 * (_B // 2), _B // 2)], sems[4 + s2])
        pltpu.async_copy(em_hbm.at[pl.ds(bi, 1)],
                         em_v.at[pl.ds(s2, 1)], sems[4 + s2])

    def _wait_eem(s2):
        pltpu.make_async_copy(ee_hbm.at[pl.ds(0, _B // 2)],
                              ee_v.at[pl.ds(s2 * (_B // 2), _B // 2)],
                              sems[4 + s2]).wait()
        pltpu.make_async_copy(em_hbm.at[pl.ds(0, 1)],
                              em_v.at[pl.ds(s2, 1)], sems[4 + s2]).wait()

    def _issue_gather(s2, b4):
        pltpu.async_copy(h_hbm.at[src_v.at[b4]],
                         hr_v.at[pl.ds(s2 * _B, _B)], sems[6 + s2])

    def _wait_gather(s2):
        pltpu.make_async_copy(h_hbm.at[pl.ds(0, _B)],
                              hr_v.at[pl.ds(s2 * _B, _B)], sems[6 + s2]).wait()

    def _issue_scatter(s2, b4):
        pltpu.async_copy(hr_v.at[pl.ds(s2 * _B, _B)],
                         agg_sh.at[dst_v.at[b4]], sems[8 + s2], add=True)

    def _wait_scatter(s2):
        pltpu.make_async_copy(hr_v.at[pl.ds(s2 * _B, _B)],
                              agg_sh.at[pl.ds(0, _B)], sems[8 + s2]).wait()

    def _compute(k, s2):
        valid = (k * _NW + wid) < _NBT
        vmask = jnp.full((16,), jnp.where(valid, 1.0, 0.0), jnp.float32)
        for j in range(8):
            em_v[s2, pl.ds(16 * j, 16)] = em_v[s2, pl.ds(16 * j, 16)] * vmask

        @plsc.parallel_loop(0, _B)
        def _row(i):
            mv = em_v[s2, pl.ds((i // 16) * 16, 16)]
            lane = jnp.full((16,), 0, jnp.int32) + (i % 16)
            m = lax.gather(
                mv, lane[:, None],
                lax.GatherDimensionNumbers(
                    offset_dims=(), collapsed_slice_dims=(0,),
                    start_index_map=(0,)),
                (1,), mode=lax.GatherScatterMode.PROMISE_IN_BOUNDS)
            r = s2 * _B + i
            r2 = s2 * (_B // 2) + i // 2
            c0 = 64 * (i % 2)
            for q in range(4):
                ew = ee_v[r2, pl.ds(c0 + 16 * q, 16)]
                ea = lax.bitcast_convert_type(ew << 16, jnp.float32)
                eb = lax.bitcast_convert_type(ew, jnp.float32)
                for half, ex in ((0, ea), (1, eb)):
                    j8 = 2 * q + half
                    v = hr_v[r, pl.ds(16 * j8, 16)] + ex
                    t = v * (-1.702)
                    hr_v[r, pl.ds(16 * j8, 16)] = (v * m) / (1.0 + jnp.exp(t))

    NB = _MAXB

    _issue_lin(0, 0)
    _issue_lin(1, 1)
    _issue_eem(0, 0)
    _issue_eem(1, 1)
    _wait_lin(0)
    _issue_gather(0, 0)

    def _outer(k0, _):
        for b in range(4):
            k = k0 * 4 + b
            s2 = b & 1

            @pl.when(k < NB)
            def _():
                @pl.when(k + 1 < NB)
                def _():
                    _wait_lin((b + 1) % 4)

                    @pl.when(k >= 1)
                    def _():
                        _wait_scatter(1 - s2)
                    _issue_gather(1 - s2, (b + 1) % 4)

                @pl.when(k + 2 < NB)
                def _():
                    _issue_lin(k + 2, (b + 2) % 4)

                _wait_eem(s2)
                _wait_gather(s2)
                _compute(k, s2)
                _issue_scatter(s2, b)

                @pl.when(k + 2 < NB)
                def _():
                    _issue_eem(k + 2, s2)
        return 0
    lax.fori_loop(0, (NB + 3) // 4, _outer, 0)

    _wait_scatter((NB - 2) % 2)
    _wait_scatter((NB - 1) % 2)

    plsc.subcore_barrier()
    pltpu.sync_copy(agg_sh.at[pl.ds(r0, _SPR)],
                    out_hbm.at[cid, pl.ds(r0, _SPR)])

    @pl.when(sid == _NS - 1)
    def _():
        pltpu.sync_copy(agg_sh.at[pl.ds(_NS * _SPR, _TAIL)],
                        out_hbm.at[cid, pl.ds(_NS * _SPR, _TAIL)])


@functools.partial(jax.jit, static_argnames=())
def _edge_call(h, eeP, srcR, dstR, emR):
    mesh = plsc.VectorSubcoreMesh(core_axis_name="c", subcore_axis_name="s")
    f = pl.kernel(
        _edge_body,
        out_type=jax.ShapeDtypeStruct((_NC, N, H), jnp.float32),
        mesh=mesh,
        scratch_types=[
            pltpu.VMEM((4, 128), jnp.int32),
            pltpu.VMEM((4, 128), jnp.int32),
            pltpu.VMEM((2, 128), jnp.float32),
            pltpu.VMEM((_B, H), jnp.int32),
            pltpu.VMEM((2 * _B, H), jnp.float32),
            pltpu.VMEM_SHARED((N, H), jnp.float32),
        ] + [pltpu.SemaphoreType.DMA] * 10,
    )
    return f(h, eeP, srcR, dstR, emR)


def _gelu_tc(v):
    return 0.5 * v * (1.0 + lax.erf(v * 0.7071067811865476))


_BE = 6400


def _ee_body(ea_ref, wb_ref, bb_ref, o_ref):
    t = jnp.dot(ea_ref[...], wb_ref[...],
                preferred_element_type=jnp.float32) + bb_ref[...]
    a = jnp.concatenate([t[:, 0:16], t[:, 32:48], t[:, 64:80], t[:, 96:112]],
                        axis=1)
    b = jnp.concatenate([t[:, 16:32], t[:, 48:64], t[:, 80:96], t[:, 112:128]],
                        axis=1)
    a16 = lax.bitcast_convert_type(a.astype(jnp.bfloat16), jnp.uint16)
    b16 = lax.bitcast_convert_type(b.astype(jnp.bfloat16), jnp.uint16)
    o_ref[...] = ((b16.astype(jnp.uint32) << 16)
                  | a16.astype(jnp.uint32)).astype(jnp.int32)


def _ee_call(edge_attr, Wb, bb2):
    return pl.pallas_call(
        _ee_body,
        grid=(E // _BE,),
        in_specs=[
            pl.BlockSpec((_BE, DE), lambda i: (i, 0)),
            pl.BlockSpec((DE, H), lambda i: (0, 0)),
            pl.BlockSpec((1, H), lambda i: (0, 0)),
        ],
        out_specs=pl.BlockSpec((_BE, H // 2), lambda i: (i, 0)),
        out_shape=jax.ShapeDtypeStruct((E, H // 2), jnp.int32),
    )(edge_attr, Wb, bb2)


_BN = 2000


def _mlp_body(eps_ref, h_ref, a_ref, w1_ref, b1_ref, w2_ref, b2_ref,
              o_ref):
    t = eps_ref[0] * h_ref[...] + a_ref[0] + a_ref[1]
    u = _gelu_tc(jnp.dot(t, w1_ref[...], preferred_element_type=jnp.float32)
                 + b1_ref[...])
    v = jnp.dot(u, w2_ref[...], preferred_element_type=jnp.float32) + b2_ref[...]
    o_ref[...] = _gelu_tc(v)


def _mlp_call(epsp, h, parts, W1, b12, W2, b22):
    return pl.pallas_call(
        _mlp_body,
        grid=(N // _BN,),
        in_specs=[
            pl.BlockSpec(memory_space=pltpu.SMEM),
            pl.BlockSpec((_BN, H), lambda i: (i, 0)),
            pl.BlockSpec((_NC, _BN, H), lambda i: (0, i, 0)),
            pl.BlockSpec((H, H), lambda i: (0, 0)),
            pl.BlockSpec((1, H), lambda i: (0, 0)),
            pl.BlockSpec((H, H), lambda i: (0, 0)),
            pl.BlockSpec((1, H), lambda i: (0, 0)),
        ],
        out_specs=pl.BlockSpec((_BN, H), lambda i: (i, 0)),
        out_shape=jax.ShapeDtypeStruct((N, H), jnp.float32),
    )(epsp, h, parts, W1, b12, W2, b22)


def _pool_body(eps_ref, h_ref, a_ref, w1_ref, b1_ref, w2_ref, b2_ref,
               b_ref, m_ref, m1_ref, mb1_ref, m2_ref, mb2_ref,
               p_ref, num_sc, den_sc):
    i = pl.program_id(0)

    @pl.when(i == 0)
    def _():
        num_sc[...] = jnp.zeros_like(num_sc)
        den_sc[...] = jnp.zeros_like(den_sc)

    t0 = eps_ref[0] * h_ref[...] + a_ref[0] + a_ref[1]
    u0 = _gelu_tc(jnp.dot(t0, w1_ref[...], preferred_element_type=jnp.float32)
                  + b1_ref[...])
    hh = _gelu_tc(jnp.dot(u0, w2_ref[...], preferred_element_type=jnp.float32)
                  + b2_ref[...])
    t = jnp.dot(_gelu_tc(jnp.dot(hh, m1_ref[...],
                                 preferred_element_type=jnp.float32)
                         + mb1_ref[...]),
                m2_ref[...], preferred_element_type=jnp.float32) + mb2_ref[...]
    b = b_ref[0, 0, :]
    g = lax.broadcasted_iota(jnp.int32, (_BN, NG), 1)
    oh = (b[:, None] == g).astype(jnp.float32)
    mask = m_ref[0, 0, :][:, None]
    num_sc[...] += lax.dot_general(oh, t * mask, (((0,), (0,)), ((), ())))
    den_sc[...] += lax.dot_general(
        oh, jnp.broadcast_to(mask, (_BN, NG)), (((0,), (0,)), ((), ())))

    @pl.when(i == pl.num_programs(0) - 1)
    def _():
        p_ref[...] = num_sc[...] / (den_sc[...] + 1e-7)


def _pool_call(epsp, h, parts, W1, b12, W2, b22, batch3, mask3,
               M1, mb12, M2, mb22):
    return pl.pallas_call(
        _pool_body,
        grid=(N // _BN,),
        in_specs=[
            pl.BlockSpec(memory_space=pltpu.SMEM),
            pl.BlockSpec((_BN, H), lambda i: (i, 0)),
            pl.BlockSpec((_NC, _BN, H), lambda i: (0, i, 0)),
            pl.BlockSpec((H, H), lambda i: (0, 0)),
            pl.BlockSpec((1, H), lambda i: (0, 0)),
            pl.BlockSpec((H, H), lambda i: (0, 0)),
            pl.BlockSpec((1, H), lambda i: (0, 0)),
            pl.BlockSpec((1, 1, _BN), lambda i: (i, 0, 0)),
            pl.BlockSpec((1, 1, _BN), lambda i: (i, 0, 0)),
            pl.BlockSpec((H, H), lambda i: (0, 0)),
            pl.BlockSpec((1, H), lambda i: (0, 0)),
            pl.BlockSpec((H, H), lambda i: (0, 0)),
            pl.BlockSpec((1, H), lambda i: (0, 0)),
        ],
        out_specs=pl.BlockSpec((NG, H), lambda i: (0, 0)),
        out_shape=jax.ShapeDtypeStruct((NG, H), jnp.float32),
        scratch_shapes=[
            pltpu.VMEM((NG, H), jnp.float32),
            pltpu.VMEM((NG, NG), jnp.float32),
        ],
    )(epsp, h, parts, W1, b12, W2, b22, batch3, mask3, M1, mb12, M2, mb22)


def kernel(x, batch, edge_index, edge_attr, node_mask, edge_mask,
           conv_W1, conv_b1, conv_W2, conv_b2, eps, Wb, bb, M1, mb1, M2, mb2):
    srcR = edge_index[0].reshape(E // 128, 128)
    dstR = edge_index[1].reshape(E // 128, 128)
    emR = edge_mask.reshape(E // 128, 128)

    eeW = _ee_call(edge_attr, Wb, bb.reshape(1, H)).reshape(E // 2, H)

    h = x
    for i in range(L - 1):
        parts = _edge_call(h, eeW, srcR, dstR, emR)
        h = _mlp_call((1.0 + eps[i]).reshape(1), h, parts,
                      conv_W1[i], conv_b1[i].reshape(1, H),
                      conv_W2[i], conv_b2[i].reshape(1, H))

    parts = _edge_call(h, eeW, srcR, dstR, emR)
    return _pool_call((1.0 + eps[L - 1]).reshape(1), h, parts,
                      conv_W1[L - 1], conv_b1[L - 1].reshape(1, H),
                      conv_W2[L - 1], conv_b2[L - 1].reshape(1, H),
                      batch.reshape(N // _BN, 1, _BN),
                      node_mask.reshape(N // _BN, 1, _BN),
                      M1, mb1.reshape(1, H), M2, mb2.reshape(1, H))
